# baseline (speedup 1.0000x reference)
import functools

import jax
import jax.numpy as jnp
from jax.experimental import pallas as pl
from jax.experimental.pallas import tpu as pltpu
from jax.experimental.pallas import tpu_sc as plsc

_E = 8
_HB = 2048
_LANES = 16


def _logits_kernel(x_ref, wr_ref, lt_ref):
    logits = jnp.dot(x_ref[...], wr_ref[...],
                     preferred_element_type=jnp.float32)
    lt_ref[...] = logits.T


def _sc_router(logits_t):
    e, t = logits_t.shape
    n_chunks = t // _LANES
    mesh = plsc.VectorSubcoreMesh(core_axis_name="c", subcore_axis_name="s")

    @pl.kernel(
        out_type=jax.ShapeDtypeStruct((e, t), jnp.float32),
        mesh=mesh,
        scratch_types=[
            pltpu.VMEM((e, _LANES), jnp.float32),
            pltpu.VMEM((e, _LANES), jnp.float32),
            pltpu.SemaphoreType.DMA,
        ],
    )
    def _k(lt_hbm, gt_hbm, lt_vmem, gt_vmem, sem):
        c = jax.lax.axis_index("c")
        s = jax.lax.axis_index("s")
        n_per_core = n_chunks // 2
        chunk = c * n_per_core + s

        @pl.when(s < n_per_core)
        def _():
            base = chunk * _LANES
            copies = [
                pltpu.make_async_copy(lt_hbm.at[i, pl.ds(base, _LANES)],
                                      lt_vmem.at[i], sem)
                for i in range(e)
            ]
            for cp in copies:
                cp.start()
            for cp in copies:
                cp.wait()
            rows = [lt_vmem[i, :] for i in range(e)]
            m = rows[0]
            for r in rows[1:]:
                m = jnp.maximum(m, r)
            ex = [jnp.exp(r - m) for r in rows]
            tot = ex[0]
            for r in ex[1:]:
                tot = tot + r
            probs = [r / tot for r in ex]
            one = jnp.full((_LANES,), 1.0, jnp.float32)
            zero = jnp.zeros((_LANES,), jnp.float32)
            for i in range(e):
                cnt = zero
                for j in range(e):
                    if j == i:
                        continue
                    beats = (probs[j] >= probs[i]) if j < i else (
                        probs[j] > probs[i])
                    cnt = cnt + jnp.where(beats, one, zero)
                gt_vmem[i, :] = jnp.where(cnt < 2.0, probs[i], zero)
            out_copies = [
                pltpu.make_async_copy(gt_vmem.at[i],
                                      gt_hbm.at[i, pl.ds(base, _LANES)], sem)
                for i in range(e)
            ]
            for cp in out_copies:
                cp.start()
            for cp in out_copies:
                cp.wait()

    return _k(logits_t)


def _ffn_kernel(x_ref, wv_ref, w_ref, w1_ref, y_ref, acc_ref, *, nh):
    h = pl.program_id(1)

    @pl.when(h == 0)
    def _init():
        acc_ref[...] = jnp.zeros_like(acc_ref)

    x = x_ref[...].astype(jnp.bfloat16)
    wv = wv_ref[0].astype(jnp.bfloat16)
    w = w_ref[0].astype(jnp.bfloat16)
    w1 = w1_ref[0].astype(jnp.bfloat16)
    v = jnp.dot(x, wv, preferred_element_type=jnp.float32)
    g = jax.nn.gelu(jnp.dot(x, w, preferred_element_type=jnp.float32))
    p = (v * g).astype(jnp.bfloat16)
    acc_ref[...] += jnp.dot(p, w1, preferred_element_type=jnp.float32)

    @pl.when(h == nh - 1)
    def _fin():
        y_ref[0] = acc_ref[...].astype(jnp.bfloat16)


def _combine_kernel(gt_ref, y_ref, o_ref):
    gates = gt_ref[...].T
    lane = jax.lax.broadcasted_iota(jnp.int32, gates.shape, 1)
    acc = jnp.zeros(o_ref.shape, jnp.float32)
    for e in range(_E):
        gate = jnp.sum(jnp.where(lane == e, gates, 0.0), axis=1,
                       keepdims=True)
        acc += y_ref[e].astype(jnp.float32) * gate
    o_ref[...] = acc.astype(jnp.bfloat16)


@jax.jit
def kernel(inputs, padding_mask, w_router, w_v, w, w1):
    B, S, D = inputs.shape
    T = B * S
    H = w_v.shape[2]
    nh = H // _HB
    x = inputs.reshape(T, D)

    logits_t = pl.pallas_call(
        _logits_kernel,
        out_shape=jax.ShapeDtypeStruct((_E, T), jnp.float32),
    )(x, w_router)

    gt = _sc_router(logits_t)

    y = pl.pallas_call(
        functools.partial(_ffn_kernel, nh=nh),
        grid=(_E, nh),
        in_specs=[
            pl.BlockSpec((T, D), lambda e, h: (0, 0)),
            pl.BlockSpec((1, D, _HB), lambda e, h: (e, 0, h)),
            pl.BlockSpec((1, D, _HB), lambda e, h: (e, 0, h)),
            pl.BlockSpec((1, _HB, D), lambda e, h: (e, h, 0)),
        ],
        out_specs=pl.BlockSpec((1, T, D), lambda e, h: (e, 0, 0)),
        out_shape=jax.ShapeDtypeStruct((_E, T, D), jnp.bfloat16),
        scratch_shapes=[
            pltpu.VMEM((T, D), jnp.float32),
        ],
    )(x, w_v, w, w1)

    out = pl.pallas_call(
        _combine_kernel,
        out_shape=jax.ShapeDtypeStruct((T, D), jnp.bfloat16),
    )(gt, y)
    return out.reshape(B, S, D)


# speedup vs baseline: 1.0038x; 1.0038x over previous
import functools

import jax
import jax.numpy as jnp
from jax.experimental import pallas as pl
from jax.experimental.pallas import tpu as pltpu
from jax.experimental.pallas import tpu_sc as plsc

_E = 8
_HB = 2048
_LANES = 16


def _logits_kernel(x_ref, wr_ref, lt_ref):
    logits = jnp.dot(x_ref[...], wr_ref[...],
                     preferred_element_type=jnp.float32)
    lt_ref[...] = logits.T


def _sc_router(logits_t):
    e, t = logits_t.shape
    n_chunks = t // _LANES
    mesh = plsc.VectorSubcoreMesh(core_axis_name="c", subcore_axis_name="s")

    @pl.kernel(
        out_type=jax.ShapeDtypeStruct((e, t), jnp.float32),
        mesh=mesh,
        scratch_types=[
            pltpu.VMEM((e, _LANES), jnp.float32),
            pltpu.VMEM((e, _LANES), jnp.float32),
            pltpu.SemaphoreType.DMA,
        ],
    )
    def _k(lt_hbm, gt_hbm, lt_vmem, gt_vmem, sem):
        c = jax.lax.axis_index("c")
        s = jax.lax.axis_index("s")
        n_per_core = n_chunks // 2
        chunk = c * n_per_core + s

        @pl.when(s < n_per_core)
        def _():
            base = chunk * _LANES
            copies = [
                pltpu.make_async_copy(lt_hbm.at[i, pl.ds(base, _LANES)],
                                      lt_vmem.at[i], sem)
                for i in range(e)
            ]
            for cp in copies:
                cp.start()
            for cp in copies:
                cp.wait()
            rows = [lt_vmem[i, :] for i in range(e)]
            m = rows[0]
            for r in rows[1:]:
                m = jnp.maximum(m, r)
            ex = [jnp.exp(r - m) for r in rows]
            tot = ex[0]
            for r in ex[1:]:
                tot = tot + r
            probs = [r / tot for r in ex]
            one = jnp.full((_LANES,), 1.0, jnp.float32)
            zero = jnp.zeros((_LANES,), jnp.float32)
            for i in range(e):
                cnt = zero
                for j in range(e):
                    if j == i:
                        continue
                    beats = (probs[j] >= probs[i]) if j < i else (
                        probs[j] > probs[i])
                    cnt = cnt + jnp.where(beats, one, zero)
                gt_vmem[i, :] = jnp.where(cnt < 2.0, probs[i], zero)
            out_copies = [
                pltpu.make_async_copy(gt_vmem.at[i],
                                      gt_hbm.at[i, pl.ds(base, _LANES)], sem)
                for i in range(e)
            ]
            for cp in out_copies:
                cp.start()
            for cp in out_copies:
                cp.wait()

    return _k(logits_t)


def _ffn_kernel(x_ref, wva_ref, wvb_ref, wa_ref, wb_ref, w1a_ref, w1b_ref,
                y_ref, acc_ref, *, nh):
    h = pl.program_id(1)

    @pl.when(h == 0)
    def _init():
        acc_ref[...] = jnp.zeros_like(acc_ref)

    x = x_ref[...].astype(jnp.bfloat16)
    y = jnp.zeros(acc_ref.shape, jnp.float32)
    for wv_ref, w_ref, w1_ref in ((wva_ref, wa_ref, w1a_ref),
                                  (wvb_ref, wb_ref, w1b_ref)):
        wv = wv_ref[0].astype(jnp.bfloat16)
        w = w_ref[0].astype(jnp.bfloat16)
        w1 = w1_ref[0].astype(jnp.bfloat16)
        v = jnp.dot(x, wv, preferred_element_type=jnp.float32)
        g = jax.nn.gelu(jnp.dot(x, w, preferred_element_type=jnp.float32))
        p = (v * g).astype(jnp.bfloat16)
        y += jnp.dot(p, w1, preferred_element_type=jnp.float32)
    acc_ref[...] += y

    @pl.when(h == nh - 1)
    def _fin():
        y_ref[0] = acc_ref[...].astype(jnp.bfloat16)


def _combine_kernel(gt_ref, y_ref, o_ref):
    gates = gt_ref[...].T
    lane = jax.lax.broadcasted_iota(jnp.int32, gates.shape, 1)
    acc = jnp.zeros(o_ref.shape, jnp.float32)
    for e in range(_E):
        gate = jnp.sum(jnp.where(lane == e, gates, 0.0), axis=1,
                       keepdims=True)
        acc += y_ref[e].astype(jnp.float32) * gate
    o_ref[...] = acc.astype(jnp.bfloat16)


@jax.jit
def kernel(inputs, padding_mask, w_router, w_v, w, w1):
    B, S, D = inputs.shape
    T = B * S
    H = w_v.shape[2]
    nh = H // _HB
    x = inputs.reshape(T, D)

    logits_t = pl.pallas_call(
        _logits_kernel,
        out_shape=jax.ShapeDtypeStruct((_E, T), jnp.float32),
    )(x, w_router)

    gt = _sc_router(logits_t)

    hs = _HB // 2
    y = pl.pallas_call(
        functools.partial(_ffn_kernel, nh=nh),
        grid=(_E, nh),
        in_specs=[
            pl.BlockSpec((T, D), lambda e, h: (0, 0)),
            pl.BlockSpec((1, D, hs), lambda e, h: (e, 0, 2 * h)),
            pl.BlockSpec((1, D, hs), lambda e, h: (e, 0, 2 * h + 1)),
            pl.BlockSpec((1, D, hs), lambda e, h: (e, 0, 2 * h)),
            pl.BlockSpec((1, D, hs), lambda e, h: (e, 0, 2 * h + 1)),
            pl.BlockSpec((1, hs, D), lambda e, h: (e, 2 * h, 0)),
            pl.BlockSpec((1, hs, D), lambda e, h: (e, 2 * h + 1, 0)),
        ],
        out_specs=pl.BlockSpec((1, T, D), lambda e, h: (e, 0, 0)),
        out_shape=jax.ShapeDtypeStruct((_E, T, D), jnp.bfloat16),
        scratch_shapes=[
            pltpu.VMEM((T, D), jnp.float32),
        ],
    )(x, w_v, w_v, w, w, w1, w1)

    out = pl.pallas_call(
        _combine_kernel,
        out_shape=jax.ShapeDtypeStruct((T, D), jnp.bfloat16),
    )(gt, y)
    return out.reshape(B, S, D)
